# baseline (speedup 1.0000x reference)
import functools

import jax
import jax.numpy as jnp
from jax import lax
from jax.experimental import pallas as pl
from jax.experimental.pallas import tpu as pltpu
from jax.experimental.pallas import tpu_sc as plsc

N_ROWS = 16384
N_COLS = 256
NC = 2
NS = 16
L = 16
NW = NC * NS
SC_ROWS = 8192
TC_ROWS = N_ROWS - SC_ROWS
ROWS_PER_W = SC_ROWS // NW
G = 64
N_GROUPS = ROWS_PER_W // G
TC_BR = 1024


def _sc_excl_cumsum(x):
    mesh = plsc.VectorSubcoreMesh(core_axis_name="c", subcore_axis_name="s")

    @functools.partial(
        pl.kernel,
        mesh=mesh,
        out_type=jax.ShapeDtypeStruct((N_ROWS, N_COLS), jnp.float32),
        scratch_types=[
            pltpu.VMEM((G, N_COLS), jnp.float32),
            pltpu.VMEM((G, N_COLS), jnp.float32),
            pltpu.VMEM((G, N_COLS), jnp.float32),
            pltpu.VMEM((G, N_COLS), jnp.float32),
            pltpu.SemaphoreType.DMA,
            pltpu.SemaphoreType.DMA,
            pltpu.SemaphoreType.DMA,
            pltpu.SemaphoreType.DMA,
        ],
        compiler_params=pltpu.CompilerParams(needs_layout_passes=False),
    )
    def k(x_hbm, out_hbm, ib0, ib1, ob0, ob1, si0, si1, so0, so1):
        ibufs, obufs = (ib0, ib1), (ob0, ob1)
        sins, souts = (si0, si1), (so0, so1)
        wid = lax.axis_index("s") * NC + lax.axis_index("c")
        row0 = wid * ROWS_PER_W
        riota = lax.iota(jnp.int32, L)
        sg_rows = [riota + sg * L for sg in range(G // L)]

        def in_copy(g):
            r0 = row0 + g * G
            return pltpu.make_async_copy(
                x_hbm.at[pl.ds(r0, G), :], ibufs[g % 2], sins[g % 2])

        def out_copy(g):
            r0 = row0 + g * G
            return pltpu.make_async_copy(
                obufs[g % 2], out_hbm.at[pl.ds(r0, G), :], souts[g % 2])

        def compute(ibuf, obuf):
            def masked_step(t, accs):
                m = (riota <= t) & (t < riota + N_COLS)
                col = t - riota
                out = []
                for rows, acc in zip(sg_rows, accs):
                    v = plsc.load_gather(ibuf, [rows, col], mask=m)
                    plsc.store_scatter(obuf, [rows, col], acc, mask=m)
                    out.append(acc + jnp.where(m, v, 0.0))
                return tuple(out)

            def step(t, accs):
                col = t - riota
                out = []
                for rows, acc in zip(sg_rows, accs):
                    v = plsc.load_gather(ibuf, [rows, col])
                    plsc.store_scatter(obuf, [rows, col], acc)
                    out.append(acc + v)
                return tuple(out)

            zero = jnp.zeros((L,), jnp.float32)
            accs = tuple(zero for _ in sg_rows)
            accs = lax.fori_loop(0, L, masked_step, accs)
            accs = lax.fori_loop(L, N_COLS, step, accs, unroll=8)
            lax.fori_loop(N_COLS, N_COLS + L - 1, masked_step, accs)

        in_copy(0).start()
        for g in range(N_GROUPS):
            if g + 1 < N_GROUPS:
                in_copy(g + 1).start()
            in_copy(g).wait()
            if g >= 2:
                out_copy(g - 2).wait()
            compute(ibufs[g % 2], obufs[g % 2])
            out_copy(g).start()
        out_copy(N_GROUPS - 2).wait()
        out_copy(N_GROUPS - 1).wait()

    return k(x)


def _tc_excl_cumsum_tail(x):
    def body(x_ref, o_ref):
        r = lax.broadcasted_iota(jnp.int32, (N_COLS, N_COLS), 0)
        c = lax.broadcasted_iota(jnp.int32, (N_COLS, N_COLS), 1)
        tri = (r < c).astype(jnp.bfloat16)
        o_ref[...] = jnp.dot(x_ref[...].astype(jnp.bfloat16), tri,
                             preferred_element_type=jnp.float32)

    return pl.pallas_call(
        body,
        grid=(TC_ROWS // TC_BR,),
        in_specs=[pl.BlockSpec((TC_BR, N_COLS),
                               lambda i: (SC_ROWS // TC_BR + i, 0))],
        out_specs=pl.BlockSpec((TC_BR, N_COLS), lambda i: (i, 0)),
        out_shape=jax.ShapeDtypeStruct((TC_ROWS, N_COLS), jnp.float32),
    )(x)


def kernel(x):
    sc_out = _sc_excl_cumsum(x)
    tc_out = _tc_excl_cumsum_tail(x)
    return lax.dynamic_update_slice(sc_out, tc_out, (SC_ROWS, 0))


# speedup vs baseline: 1.0960x; 1.0960x over previous
import functools

import jax
import jax.numpy as jnp
from jax import lax
from jax.experimental import pallas as pl
from jax.experimental.pallas import tpu as pltpu
from jax.experimental.pallas import tpu_sc as plsc

N_ROWS = 16384
N_COLS = 256
NC = 2
NS = 16
L = 16
NW = NC * NS
SC_ROWS = 6144
TC_ROWS = N_ROWS - SC_ROWS
ROWS_PER_W = SC_ROWS // NW
G = 64
N_GROUPS = ROWS_PER_W // G
TC_BR = 1024


def _sc_excl_cumsum(x):
    mesh = plsc.VectorSubcoreMesh(core_axis_name="c", subcore_axis_name="s")

    @functools.partial(
        pl.kernel,
        mesh=mesh,
        out_type=jax.ShapeDtypeStruct((SC_ROWS, N_COLS), jnp.float32),
        scratch_types=[
            pltpu.VMEM((G, N_COLS), jnp.float32),
            pltpu.VMEM((G, N_COLS), jnp.float32),
            pltpu.VMEM((G, N_COLS), jnp.float32),
            pltpu.VMEM((G, N_COLS), jnp.float32),
            pltpu.SemaphoreType.DMA,
            pltpu.SemaphoreType.DMA,
            pltpu.SemaphoreType.DMA,
            pltpu.SemaphoreType.DMA,
        ],
        compiler_params=pltpu.CompilerParams(needs_layout_passes=False),
    )
    def k(x_hbm, out_hbm, ib0, ib1, ob0, ob1, si0, si1, so0, so1):
        ibufs, obufs = (ib0, ib1), (ob0, ob1)
        sins, souts = (si0, si1), (so0, so1)
        wid = lax.axis_index("s") * NC + lax.axis_index("c")
        row0 = wid * ROWS_PER_W
        riota = lax.iota(jnp.int32, L)
        sg_rows = [riota + sg * L for sg in range(G // L)]

        def in_copy(g):
            r0 = row0 + g * G
            return pltpu.make_async_copy(
                x_hbm.at[pl.ds(r0, G), :], ibufs[g % 2], sins[g % 2])

        def out_copy(g):
            r0 = row0 + g * G
            return pltpu.make_async_copy(
                obufs[g % 2], out_hbm.at[pl.ds(r0, G), :], souts[g % 2])

        def compute(ibuf, obuf):
            def masked_step(t, accs):
                m = (riota <= t) & (t < riota + N_COLS)
                col = t - riota
                out = []
                for rows, acc in zip(sg_rows, accs):
                    v = plsc.load_gather(ibuf, [rows, col], mask=m)
                    plsc.store_scatter(obuf, [rows, col], acc, mask=m)
                    out.append(acc + jnp.where(m, v, 0.0))
                return tuple(out)

            def step(t, accs):
                col = t - riota
                out = []
                for rows, acc in zip(sg_rows, accs):
                    v = plsc.load_gather(ibuf, [rows, col])
                    plsc.store_scatter(obuf, [rows, col], acc)
                    out.append(acc + v)
                return tuple(out)

            zero = jnp.zeros((L,), jnp.float32)
            accs = tuple(zero for _ in sg_rows)
            accs = lax.fori_loop(0, L, masked_step, accs)
            accs = lax.fori_loop(L, N_COLS, step, accs, unroll=8)
            lax.fori_loop(N_COLS, N_COLS + L - 1, masked_step, accs)

        in_copy(0).start()
        for g in range(N_GROUPS):
            if g + 1 < N_GROUPS:
                in_copy(g + 1).start()
            in_copy(g).wait()
            if g >= 2:
                out_copy(g - 2).wait()
            compute(ibufs[g % 2], obufs[g % 2])
            out_copy(g).start()
        out_copy(N_GROUPS - 2).wait()
        out_copy(N_GROUPS - 1).wait()

    return k(x)


def _tc_excl_cumsum_tail(x):
    def body(x_ref, o_ref):
        r = lax.broadcasted_iota(jnp.int32, (N_COLS, N_COLS), 0)
        c = lax.broadcasted_iota(jnp.int32, (N_COLS, N_COLS), 1)
        tri = (r < c).astype(jnp.float32)
        o_ref[...] = jnp.dot(x_ref[...], tri,
                             preferred_element_type=jnp.float32)

    return pl.pallas_call(
        body,
        grid=(TC_ROWS // TC_BR,),
        in_specs=[pl.BlockSpec((TC_BR, N_COLS),
                               lambda i: (SC_ROWS // TC_BR + i, 0))],
        out_specs=pl.BlockSpec((TC_BR, N_COLS),
                               lambda i: (SC_ROWS // TC_BR + i, 0)),
        out_shape=jax.ShapeDtypeStruct((N_ROWS, N_COLS), jnp.float32),
    )(x)


def kernel(x):
    sc_out = _sc_excl_cumsum(x)
    tc_out = _tc_excl_cumsum_tail(x)
    return lax.dynamic_update_slice(tc_out, sc_out, (0, 0))
